# baseline (device time: 16927 ns/iter reference)
import jax
import jax.numpy as jnp
from jax import lax
from jax.experimental import pallas as pl
from jax.experimental.pallas import tpu as pltpu

N_DEV = 16
KW = 4
HALO = KW - 1
PAD = 8
NSEQ = 2


def _silu(v):
    return v * (0.5 * jnp.tanh(0.5 * v) + 0.5)


def kernel(x, k):
    b, s, c = x.shape
    ch = s // NSEQ
    nchunks = b * NSEQ
    order = [(g, 1) for g in range(b)] + [(g, 0) for g in range(b)]

    def body(x_ref, k_ref, out_ref, bufs, obufs, halo_ref, send_buf,
             in_sems, out_sems, stage_sem, send_sem, recv_sem):
        my = lax.axis_index("i")
        left = jnp.where(my > 0, my - 1, N_DEV - 1)
        right = jnp.where(my < N_DEV - 1, my + 1, 0)

        @pl.when(my > 0)
        def _():
            pl.semaphore_signal(
                pltpu.get_barrier_semaphore(), inc=1,
                device_id=(left,), device_id_type=pl.DeviceIdType.MESH,
            )

        stage = pltpu.make_async_copy(
            x_ref.at[:, pl.ds(s - PAD, PAD), :], send_buf, stage_sem,
        )
        stage.start()

        in_copies = []
        for j, (g, h) in enumerate(order):
            if h == 0:
                cp = pltpu.make_async_copy(
                    x_ref.at[g, pl.ds(0, ch), :],
                    bufs.at[j, pl.ds(PAD, ch), :],
                    in_sems.at[j],
                )
            else:
                cp = pltpu.make_async_copy(
                    x_ref.at[g, pl.ds(h * ch - PAD, ch + PAD), :],
                    bufs.at[j],
                    in_sems.at[j],
                )
            cp.start()
            in_copies.append(cp)

        rdma = pltpu.make_async_remote_copy(
            src_ref=send_buf,
            dst_ref=halo_ref,
            send_sem=send_sem,
            recv_sem=recv_sem,
            device_id=(right,),
            device_id_type=pl.DeviceIdType.MESH,
        )

        stage.wait()

        @pl.when(my < N_DEV - 1)
        def _():
            pl.semaphore_wait(pltpu.get_barrier_semaphore(), 1)
            rdma.start()

        out_copies = []
        for j, (g, h) in enumerate(order):
            if h == 0 and g == 0:
                @pl.when(my > 0)
                def _():
                    rdma.wait_recv()

                @pl.when(my == 0)
                def _():
                    halo_ref[...] = jnp.zeros((b, PAD, c), jnp.float32)

            in_copies[j].wait()
            if h == 0:
                bufs[j, PAD - HALO:PAD, :] = halo_ref[g, PAD - HALO:, :]

            bv = bufs[j]
            acc = bv[PAD - HALO:PAD - HALO + ch, :] * k_ref[0, :]
            for t in range(1, KW):
                acc += bv[PAD - HALO + t:PAD - HALO + t + ch, :] * k_ref[t, :]
            obufs[j] = _silu(acc)

            ocp = pltpu.make_async_copy(
                obufs.at[j],
                out_ref.at[g, pl.ds(h * ch, ch), :],
                out_sems.at[j],
            )
            ocp.start()
            out_copies.append(ocp)

        for ocp in out_copies:
            ocp.wait()

        @pl.when(my < N_DEV - 1)
        def _():
            rdma.wait_send()

    return pl.pallas_call(
        body,
        out_shape=jax.ShapeDtypeStruct((b, s, c), jnp.float32),
        in_specs=[
            pl.BlockSpec(memory_space=pltpu.MemorySpace.HBM),
            pl.BlockSpec(memory_space=pltpu.VMEM),
        ],
        out_specs=pl.BlockSpec(memory_space=pltpu.MemorySpace.HBM),
        scratch_shapes=[
            pltpu.VMEM((nchunks, ch + PAD, c), jnp.float32),
            pltpu.VMEM((nchunks, ch, c), jnp.float32),
            pltpu.VMEM((b, PAD, c), jnp.float32),
            pltpu.VMEM((b, PAD, c), jnp.float32),
            pltpu.SemaphoreType.DMA((nchunks,)),
            pltpu.SemaphoreType.DMA((nchunks,)),
            pltpu.SemaphoreType.DMA,
            pltpu.SemaphoreType.DMA,
            pltpu.SemaphoreType.DMA,
        ],
        compiler_params=pltpu.CompilerParams(collective_id=0),
    )(x, k)
